# baseline (speedup 1.0000x reference)
import functools

import jax
import jax.numpy as jnp
from jax import lax
from jax.experimental import pallas as pl
from jax.experimental.pallas import tpu as pltpu
from jax.experimental.pallas import tpu_sc as plsc

_NC = 2
_NS = 16
_NW = _NC * _NS
_L = 16
_UNROLL = 4

_TH_CO = (1.947637334e-02, -9.259107687e-02, 2.116797744e-01,
          -3.424470462e-01, 5.060373501e-01, -8.486314227e-01,
          2.546477117e+00)
_AC_CO = (7.790549338e-02, -3.703643075e-01, 8.467190975e-01,
          -1.369788185e+00, 2.024149400e+00, -3.394525691e+00,
          1.018590847e+01)


def _poly(co, u):
    r = jnp.float32(co[0])
    for c in co[1:]:
        r = r * u + jnp.float32(c)
    return r


def _compute16(x, y, z):
    w = -z
    aa = jnp.abs(x)
    ab = jnp.abs(w)
    mx = jnp.maximum(jnp.maximum(aa, ab), jnp.float32(1e-30))
    t = jnp.minimum(aa, ab) / mx
    r = _poly(_TH_CO, t * t) * t
    r = jnp.where(aa > ab, jnp.float32(4.0) - r, r)
    r = jnp.where(w < 0, jnp.float32(8.0) - r, r)
    iy = jnp.where(x < 0, -r, r) + jnp.float32(7.5)

    u = jnp.maximum((jnp.float32(1.0) - y) * (jnp.float32(1.0) + y),
                    jnp.float32(1e-30))
    i = lax.bitcast_convert_type(u, jnp.int32)
    i = jnp.int32(0x5F3759DF) - lax.shift_right_logical(i, jnp.int32(1))
    h = lax.bitcast_convert_type(i, jnp.float32)
    h = h * (jnp.float32(1.5) - jnp.float32(0.5) * u * h * h)
    h = h * (jnp.float32(1.5) - jnp.float32(0.5) * u * h * h)
    s = u * h
    ay = jnp.abs(y)
    t2 = jnp.minimum(s, ay) / jnp.maximum(s, ay)
    r2 = _poly(_AC_CO, t2 * t2) * t2
    r2 = jnp.where(s > ay, jnp.float32(16.0) - r2, r2)
    r2 = jnp.where(y < 0, jnp.float32(32.0) - r2, r2)
    ix = r2 - jnp.float32(0.5)

    ix0 = (ix + jnp.float32(1.0)).astype(jnp.int32) - 1
    iy0 = (iy + jnp.float32(1.0)).astype(jnp.int32) - 1
    wx1 = ix - ix0.astype(jnp.float32)
    wy1 = iy - iy0.astype(jnp.float32)
    wx0 = jnp.float32(1.0) - wx1
    wy0 = jnp.float32(1.0) - wy1
    j = iy0 * 34 + ix0 + 35
    return j, (wy0 * wx0, wy0 * wx1, wy1 * wx0, wy1 * wx1)


def _sc_body(in_hbm, env_hbm, out_hbm, iv, tbl, ov, npw):
    wid = lax.axis_index("s") * _NC + lax.axis_index("c")
    base3 = wid * (npw * 3)
    pltpu.sync_copy(env_hbm, tbl)
    pltpu.sync_copy(in_hbm.at[pl.ds(base3, npw * 3)], iv)

    iota3 = lax.iota(jnp.int32, _L) * 3

    def body(i, carry):
        for k in range(_UNROLL):
            off = i * (_UNROLL * _L * 3) + (k * _L * 3)
            tdx = iota3 + off
            x = plsc.load_gather(iv, [tdx])
            y = plsc.load_gather(iv, [tdx + 1])
            z = plsc.load_gather(iv, [tdx + 2])
            j, (w00, w01, w10, w11) = _compute16(x, y, z)
            for c in range(3):
                jc = j + (c * 612)
                sm = (w00 * plsc.load_gather(tbl, [jc])
                      + w01 * plsc.load_gather(tbl, [jc + 1])
                      + w10 * plsc.load_gather(tbl, [jc + 34])
                      + w11 * plsc.load_gather(tbl, [jc + 35]))
                plsc.store_scatter(ov, [tdx + c], jnp.exp(sm))
        return carry

    lax.fori_loop(0, npw // (_L * _UNROLL), body, 0)
    pltpu.sync_copy(ov, out_hbm.at[pl.ds(base3, npw * 3)])


def kernel(inputs, env_map):
    n = inputs.shape[0]
    npw = n // _NW
    in_flat = inputs.reshape(-1)
    env_pad = jnp.pad(env_map[0], ((0, 0), (1, 1), (1, 1))).reshape(-1)

    mesh = plsc.VectorSubcoreMesh(
        core_axis_name="c", subcore_axis_name="s",
        num_cores=_NC, num_subcores=_NS)
    sc_call = pl.kernel(
        functools.partial(_sc_body, npw=npw),
        out_type=jax.ShapeDtypeStruct((3 * n,), jnp.float32),
        mesh=mesh,
        compiler_params=pltpu.CompilerParams(needs_layout_passes=False),
        scratch_types=(
            pltpu.VMEM((npw * 3,), jnp.float32),
            pltpu.VMEM((1836,), jnp.float32),
            pltpu.VMEM((npw * 3,), jnp.float32),
        ),
    )
    return sc_call(in_flat, env_pad).reshape(n, 3)


# speedup vs baseline: 2.4118x; 2.4118x over previous
import functools

import jax
import jax.numpy as jnp
from jax import lax
from jax.experimental import pallas as pl
from jax.experimental.pallas import tpu as pltpu
from jax.experimental.pallas import tpu_sc as plsc

_NC = 2
_NS = 16
_NW = _NC * _NS
_L = 16
_UNROLL = 4

_TH_CO = (1.947637334e-02, -9.259107687e-02, 2.116797744e-01,
          -3.424470462e-01, 5.060373501e-01, -8.486314227e-01,
          2.546477117e+00)
_AC_CO = (7.790549338e-02, -3.703643075e-01, 8.467190975e-01,
          -1.369788185e+00, 2.024149400e+00, -3.394525691e+00,
          1.018590847e+01)


def _poly(co, u):
    r = jnp.float32(co[0])
    for c in co[1:]:
        r = r * u + jnp.float32(c)
    return r


def _compute16(x, y, z):
    w = -z
    aa = jnp.abs(x)
    ab = jnp.abs(w)
    mx = jnp.maximum(jnp.maximum(aa, ab), jnp.float32(1e-30))
    t = jnp.minimum(aa, ab) / mx
    r = _poly(_TH_CO, t * t) * t
    r = jnp.where(aa > ab, jnp.float32(4.0) - r, r)
    r = jnp.where(w < 0, jnp.float32(8.0) - r, r)
    iy = jnp.where(x < 0, -r, r) + jnp.float32(7.5)

    u = jnp.maximum((jnp.float32(1.0) - y) * (jnp.float32(1.0) + y),
                    jnp.float32(1e-30))
    i = lax.bitcast_convert_type(u, jnp.int32)
    i = jnp.int32(0x5F3759DF) - lax.shift_right_logical(i, jnp.int32(1))
    h = lax.bitcast_convert_type(i, jnp.float32)
    h = h * (jnp.float32(1.5) - jnp.float32(0.5) * u * h * h)
    h = h * (jnp.float32(1.5) - jnp.float32(0.5) * u * h * h)
    s = u * h
    ay = jnp.abs(y)
    t2 = jnp.minimum(s, ay) / jnp.maximum(s, ay)
    r2 = _poly(_AC_CO, t2 * t2) * t2
    r2 = jnp.where(s > ay, jnp.float32(16.0) - r2, r2)
    r2 = jnp.where(y < 0, jnp.float32(32.0) - r2, r2)
    ix = r2 - jnp.float32(0.5)

    ix0 = (ix + jnp.float32(1.0)).astype(jnp.int32) - 1
    iy0 = (iy + jnp.float32(1.0)).astype(jnp.int32) - 1
    wx1 = ix - ix0.astype(jnp.float32)
    wy1 = iy - iy0.astype(jnp.float32)
    wx0 = jnp.float32(1.0) - wx1
    wy0 = jnp.float32(1.0) - wy1
    j = iy0 * 34 + ix0 + 35
    return j, (wy0 * wx0, wy0 * wx1, wy1 * wx0, wy1 * wx1)


def _sc_body(x_hbm, y_hbm, z_hbm, env_hbm, r_hbm, g_hbm, b_hbm,
             xv, yv, zv, tbl, rv, gv, bv, npw):
    wid = lax.axis_index("s") * _NC + lax.axis_index("c")
    base = wid * npw
    pltpu.sync_copy(env_hbm, tbl)
    pltpu.sync_copy(x_hbm.at[pl.ds(base, npw)], xv)
    pltpu.sync_copy(y_hbm.at[pl.ds(base, npw)], yv)
    pltpu.sync_copy(z_hbm.at[pl.ds(base, npw)], zv)

    def body(i, carry):
        for k in range(_UNROLL):
            sl = pl.ds(i * (_UNROLL * _L) + k * _L, _L)
            j, (w00, w01, w10, w11) = _compute16(xv[sl], yv[sl], zv[sl])
            for out_ref, coff in ((rv, 0), (gv, 612), (bv, 1224)):
                jc = j + coff
                sm = (w00 * plsc.load_gather(tbl, [jc])
                      + w01 * plsc.load_gather(tbl, [jc + 1])
                      + w10 * plsc.load_gather(tbl, [jc + 34])
                      + w11 * plsc.load_gather(tbl, [jc + 35]))
                out_ref[sl] = jnp.exp(sm)
        return carry

    lax.fori_loop(0, npw // (_L * _UNROLL), body, 0)
    pltpu.sync_copy(rv, r_hbm.at[pl.ds(base, npw)])
    pltpu.sync_copy(gv, g_hbm.at[pl.ds(base, npw)])
    pltpu.sync_copy(bv, b_hbm.at[pl.ds(base, npw)])


def kernel(inputs, env_map):
    n = inputs.shape[0]
    npw = n // _NW
    xs = inputs[:, 0]
    ys = inputs[:, 1]
    zs = inputs[:, 2]
    env_pad = jnp.pad(env_map[0], ((0, 0), (1, 1), (1, 1))).reshape(-1)

    mesh = plsc.VectorSubcoreMesh(
        core_axis_name="c", subcore_axis_name="s",
        num_cores=_NC, num_subcores=_NS)
    out_t = jax.ShapeDtypeStruct((n,), jnp.float32)
    sc_call = pl.kernel(
        functools.partial(_sc_body, npw=npw),
        out_type=(out_t, out_t, out_t),
        mesh=mesh,
        compiler_params=pltpu.CompilerParams(needs_layout_passes=False),
        scratch_types=(
            pltpu.VMEM((npw,), jnp.float32),
            pltpu.VMEM((npw,), jnp.float32),
            pltpu.VMEM((npw,), jnp.float32),
            pltpu.VMEM((1836,), jnp.float32),
            pltpu.VMEM((npw,), jnp.float32),
            pltpu.VMEM((npw,), jnp.float32),
            pltpu.VMEM((npw,), jnp.float32),
        ),
    )
    r, g, b = sc_call(xs, ys, zs, env_pad)
    return jnp.stack([r, g, b], axis=-1)


# speedup vs baseline: 2.6033x; 1.0794x over previous
import functools

import jax
import jax.numpy as jnp
from jax import lax
from jax.experimental import pallas as pl
from jax.experimental.pallas import tpu as pltpu
from jax.experimental.pallas import tpu_sc as plsc

_NC = 2
_NS = 16
_NW = _NC * _NS
_L = 16
_UNROLL = 1

_TH_CO = (1.947637334e-02, -9.259107687e-02, 2.116797744e-01,
          -3.424470462e-01, 5.060373501e-01, -8.486314227e-01,
          2.546477117e+00)
_AC_CO = (7.790549338e-02, -3.703643075e-01, 8.467190975e-01,
          -1.369788185e+00, 2.024149400e+00, -3.394525691e+00,
          1.018590847e+01)


def _poly(co, u):
    r = jnp.float32(co[0])
    for c in co[1:]:
        r = r * u + jnp.float32(c)
    return r


def _compute16(x, y, z):
    w = -z
    aa = jnp.abs(x)
    ab = jnp.abs(w)
    mx = jnp.maximum(jnp.maximum(aa, ab), jnp.float32(1e-30))
    t = jnp.minimum(aa, ab) / mx
    r = _poly(_TH_CO, t * t) * t
    r = jnp.where(aa > ab, jnp.float32(4.0) - r, r)
    r = jnp.where(w < 0, jnp.float32(8.0) - r, r)
    iy = jnp.where(x < 0, -r, r) + jnp.float32(7.5)

    u = jnp.maximum((jnp.float32(1.0) - y) * (jnp.float32(1.0) + y),
                    jnp.float32(1e-30))
    i = lax.bitcast_convert_type(u, jnp.int32)
    i = jnp.int32(0x5F3759DF) - lax.shift_right_logical(i, jnp.int32(1))
    h = lax.bitcast_convert_type(i, jnp.float32)
    h = h * (jnp.float32(1.5) - jnp.float32(0.5) * u * h * h)
    h = h * (jnp.float32(1.5) - jnp.float32(0.5) * u * h * h)
    s = u * h
    ay = jnp.abs(y)
    t2 = jnp.minimum(s, ay) / jnp.maximum(s, ay)
    r2 = _poly(_AC_CO, t2 * t2) * t2
    r2 = jnp.where(s > ay, jnp.float32(16.0) - r2, r2)
    r2 = jnp.where(y < 0, jnp.float32(32.0) - r2, r2)
    ix = r2 - jnp.float32(0.5)

    ix0 = (ix + jnp.float32(1.0)).astype(jnp.int32) - 1
    iy0 = (iy + jnp.float32(1.0)).astype(jnp.int32) - 1
    wx1 = ix - ix0.astype(jnp.float32)
    wy1 = iy - iy0.astype(jnp.float32)
    wx0 = jnp.float32(1.0) - wx1
    wy0 = jnp.float32(1.0) - wy1
    j = iy0 * 34 + ix0 + 35
    return j, (wy0 * wx0, wy0 * wx1, wy1 * wx0, wy1 * wx1)


def _sc_body(x_hbm, y_hbm, z_hbm, env_hbm, r_hbm, g_hbm, b_hbm,
             xv, yv, zv, tbl, rv, gv, bv, npw):
    wid = lax.axis_index("s") * _NC + lax.axis_index("c")
    base = wid * npw
    pltpu.sync_copy(env_hbm, tbl)
    pltpu.sync_copy(x_hbm.at[pl.ds(base, npw)], xv)
    pltpu.sync_copy(y_hbm.at[pl.ds(base, npw)], yv)
    pltpu.sync_copy(z_hbm.at[pl.ds(base, npw)], zv)

    def body(i, carry):
        for k in range(_UNROLL):
            sl = pl.ds(i * (_UNROLL * _L) + k * _L, _L)
            j, (w00, w01, w10, w11) = _compute16(xv[sl], yv[sl], zv[sl])
            for out_ref, coff in ((rv, 0), (gv, 612), (bv, 1224)):
                jc = j + coff
                sm = (w00 * plsc.load_gather(tbl, [jc])
                      + w01 * plsc.load_gather(tbl, [jc + 1])
                      + w10 * plsc.load_gather(tbl, [jc + 34])
                      + w11 * plsc.load_gather(tbl, [jc + 35]))
                out_ref[sl] = jnp.exp(sm)
        return carry

    lax.fori_loop(0, npw // (_L * _UNROLL), body, 0)
    pltpu.sync_copy(rv, r_hbm.at[pl.ds(base, npw)])
    pltpu.sync_copy(gv, g_hbm.at[pl.ds(base, npw)])
    pltpu.sync_copy(bv, b_hbm.at[pl.ds(base, npw)])


def kernel(inputs, env_map):
    n = inputs.shape[0]
    npw = n // _NW
    xs = inputs[:, 0]
    ys = inputs[:, 1]
    zs = inputs[:, 2]
    env_pad = jnp.pad(env_map[0], ((0, 0), (1, 1), (1, 1))).reshape(-1)

    mesh = plsc.VectorSubcoreMesh(
        core_axis_name="c", subcore_axis_name="s",
        num_cores=_NC, num_subcores=_NS)
    out_t = jax.ShapeDtypeStruct((n,), jnp.float32)
    sc_call = pl.kernel(
        functools.partial(_sc_body, npw=npw),
        out_type=(out_t, out_t, out_t),
        mesh=mesh,
        compiler_params=pltpu.CompilerParams(needs_layout_passes=False),
        scratch_types=(
            pltpu.VMEM((npw,), jnp.float32),
            pltpu.VMEM((npw,), jnp.float32),
            pltpu.VMEM((npw,), jnp.float32),
            pltpu.VMEM((1836,), jnp.float32),
            pltpu.VMEM((npw,), jnp.float32),
            pltpu.VMEM((npw,), jnp.float32),
            pltpu.VMEM((npw,), jnp.float32),
        ),
    )
    r, g, b = sc_call(xs, ys, zs, env_pad)
    return jnp.stack([r, g, b], axis=-1)


# speedup vs baseline: 2.8935x; 1.1115x over previous
import functools

import jax
import jax.numpy as jnp
from jax import lax
from jax.experimental import pallas as pl
from jax.experimental.pallas import tpu as pltpu
from jax.experimental.pallas import tpu_sc as plsc

_NC = 2
_NS = 16
_NW = _NC * _NS
_L = 16
_UNROLL = 2

_TH_CO = (1.947637334e-02, -9.259107687e-02, 2.116797744e-01,
          -3.424470462e-01, 5.060373501e-01, -8.486314227e-01,
          2.546477117e+00)
_AC_CO = (7.790549338e-02, -3.703643075e-01, 8.467190975e-01,
          -1.369788185e+00, 2.024149400e+00, -3.394525691e+00,
          1.018590847e+01)


def _poly(co, u):
    r = jnp.float32(co[0])
    for c in co[1:]:
        r = r * u + jnp.float32(c)
    return r


def _compute16(x, y, z):
    w = -z
    aa = jnp.abs(x)
    ab = jnp.abs(w)
    mx = jnp.maximum(jnp.maximum(aa, ab), jnp.float32(1e-30))
    t = jnp.minimum(aa, ab) / mx
    r = _poly(_TH_CO, t * t) * t
    r = jnp.where(aa > ab, jnp.float32(4.0) - r, r)
    r = jnp.where(w < 0, jnp.float32(8.0) - r, r)
    iy = jnp.where(x < 0, -r, r) + jnp.float32(7.5)

    u = jnp.maximum((jnp.float32(1.0) - y) * (jnp.float32(1.0) + y),
                    jnp.float32(1e-30))
    i = lax.bitcast_convert_type(u, jnp.int32)
    i = jnp.int32(0x5F3759DF) - lax.shift_right_logical(i, jnp.int32(1))
    h = lax.bitcast_convert_type(i, jnp.float32)
    h = h * (jnp.float32(1.5) - jnp.float32(0.5) * u * h * h)
    h = h * (jnp.float32(1.5) - jnp.float32(0.5) * u * h * h)
    s = u * h
    ay = jnp.abs(y)
    t2 = jnp.minimum(s, ay) / jnp.maximum(s, ay)
    r2 = _poly(_AC_CO, t2 * t2) * t2
    r2 = jnp.where(s > ay, jnp.float32(16.0) - r2, r2)
    r2 = jnp.where(y < 0, jnp.float32(32.0) - r2, r2)
    ix = r2 - jnp.float32(0.5)

    ix0 = (ix + jnp.float32(1.0)).astype(jnp.int32) - 1
    iy0 = (iy + jnp.float32(1.0)).astype(jnp.int32) - 1
    wx1 = ix - ix0.astype(jnp.float32)
    wy1 = iy - iy0.astype(jnp.float32)
    wx0 = jnp.float32(1.0) - wx1
    wy0 = jnp.float32(1.0) - wy1
    j = iy0 * 34 + ix0 + 35
    return j, (wy0 * wx0, wy0 * wx1, wy1 * wx0, wy1 * wx1)


def _sc_body(x_hbm, y_hbm, z_hbm, env_hbm, r_hbm, g_hbm, b_hbm,
             xv, yv, zv, tbl, rv, gv, bv, sem, npw):
    wid = lax.axis_index("s") * _NC + lax.axis_index("c")
    base = wid * npw
    cps = (pltpu.async_copy(env_hbm, tbl, sem),
           pltpu.async_copy(x_hbm.at[pl.ds(base, npw)], xv, sem),
           pltpu.async_copy(y_hbm.at[pl.ds(base, npw)], yv, sem),
           pltpu.async_copy(z_hbm.at[pl.ds(base, npw)], zv, sem))
    for c in cps:
        c.wait()

    @plsc.parallel_loop(0, npw // _L, 1, unroll=_UNROLL)
    def body(i):
        sl = pl.ds(i * _L, _L)
        j, (w00, w01, w10, w11) = _compute16(xv[sl], yv[sl], zv[sl])
        for out_ref, coff in ((rv, 0), (gv, 612), (bv, 1224)):
            jc = j + coff
            sm = (w00 * plsc.load_gather(tbl, [jc])
                  + w01 * plsc.load_gather(tbl, [jc + 1])
                  + w10 * plsc.load_gather(tbl, [jc + 34])
                  + w11 * plsc.load_gather(tbl, [jc + 35]))
            out_ref[sl] = jnp.exp(sm)

    ocps = (pltpu.async_copy(rv, r_hbm.at[pl.ds(base, npw)], sem),
            pltpu.async_copy(gv, g_hbm.at[pl.ds(base, npw)], sem),
            pltpu.async_copy(bv, b_hbm.at[pl.ds(base, npw)], sem))
    for c in ocps:
        c.wait()


def kernel(inputs, env_map):
    n = inputs.shape[0]
    npw = n // _NW
    xs = inputs[:, 0]
    ys = inputs[:, 1]
    zs = inputs[:, 2]
    env_pad = jnp.pad(env_map[0], ((0, 0), (1, 1), (1, 1))).reshape(-1)

    mesh = plsc.VectorSubcoreMesh(
        core_axis_name="c", subcore_axis_name="s",
        num_cores=_NC, num_subcores=_NS)
    out_t = jax.ShapeDtypeStruct((n,), jnp.float32)
    sc_call = pl.kernel(
        functools.partial(_sc_body, npw=npw),
        out_type=(out_t, out_t, out_t),
        mesh=mesh,
        compiler_params=pltpu.CompilerParams(needs_layout_passes=False),
        scratch_types=(
            pltpu.VMEM((npw,), jnp.float32),
            pltpu.VMEM((npw,), jnp.float32),
            pltpu.VMEM((npw,), jnp.float32),
            pltpu.VMEM((1836,), jnp.float32),
            pltpu.VMEM((npw,), jnp.float32),
            pltpu.VMEM((npw,), jnp.float32),
            pltpu.VMEM((npw,), jnp.float32),
            pltpu.SemaphoreType.DMA,
        ),
    )
    r, g, b = sc_call(xs, ys, zs, env_pad)
    return jnp.stack([r, g, b], axis=-1)
